# baseline (speedup 1.0000x reference)
import jax
import jax.numpy as jnp
from jax import lax
from jax.experimental import pallas as pl
from jax.experimental.pallas import tpu as pltpu
from jax.experimental.pallas import tpu_sc as plsc

B, F, L, D, V = 1024, 26, 20, 128, 1000
N = B * F
NW = 32
PW = N // NW
CQ = 16
IDS = CQ * L
NG = PW // CQ
LANES = 16
FH = F // 2
ZROW = FH * (V + 1)
STG = 816
SROWS = 16 * STG


def _take(vec, idx):
    dnums = lax.GatherDimensionNumbers(
        offset_dims=(), collapsed_slice_dims=(0,), start_index_map=(0,))
    return lax.gather(vec, idx[:, None], dnums, (1,),
                      mode=lax.GatherScatterMode.PROMISE_IN_BOUNDS)


def _body(ids_hbm, len_hbm, perm_hbm, table_hbm, out_hbm,
          len_v, perm_v, idsb_v, idx_v, acc_v, outc_v, shared_v,
          isems, gsems, osems):
    c = lax.axis_index("c")
    s = lax.axis_index("s")
    wid = c * 16 + s
    pstart = wid * PW

    pltpu.sync_copy(table_hbm.at[c, pl.ds(s * STG, STG)],
                    shared_v.at[pl.ds(s * STG, STG)])
    plsc.subcore_barrier()

    pltpu.sync_copy(len_hbm.at[pl.ds(pstart, PW)], len_v.at[pl.ds(0, PW)])
    pltpu.sync_copy(perm_hbm.at[pl.ds(pstart, PW)], perm_v.at[pl.ds(0, PW)])

    lane = lax.iota(jnp.int32, LANES)
    zero16 = jnp.zeros((LANES,), jnp.float32)

    for par in range(2):
        for p in range(CQ):
            for cc in range(D // LANES):
                acc_v[par, p, pl.ds(cc * LANES, LANES)] = zero16

    kbase = wid * NG

    def _fire_ids(g, b):
        pltpu.async_copy(ids_hbm.at[kbase + g], idsb_v.at[b], isems[b])

    def _mkidx(g, b):
        len16 = len_v[pl.ds(g * CQ, CQ)]
        perm16 = perm_v[pl.ds(g * CQ, CQ)]
        base16 = (lax.shift_right_logical(perm16, 10) - c * FH) * (V + 1)
        for l in range(L):
            id16 = idsb_v[b, l]
            idx = jnp.where(len16 > l, base16 + id16, ZROW)
            idx_v[b, l] = idx

    def _pool_copy(par, l):
        return pltpu.make_async_copy(
            shared_v.at[idx_v.at[par, l]], acc_v.at[par], gsems[par])

    def _fire_pool(g, par):
        m = len_v[pl.ds(g * CQ, CQ)][CQ - 1]
        for l in range(L):
            @pl.when(l < m)
            def _():
                pltpu.async_copy(
                    shared_v.at[idx_v.at[par, l]],
                    acc_v.at[par], gsems[par], add=True)

    _fire_ids(0, 0)
    pltpu.make_async_copy(ids_hbm.at[kbase], idsb_v.at[0], isems[0]).wait()
    _mkidx(0, 0)
    _fire_ids(1, 1)
    _fire_pool(0, 0)

    @pl.loop(0, NG, step=2)
    def _pool(g0):
        for par in range(2):
            g = g0 + par
            npar = (par + 1) % 2

            @pl.when(g + 1 < NG)
            def _():
                pltpu.make_async_copy(
                    ids_hbm.at[kbase + g + 1],
                    idsb_v.at[npar], isems[npar]).wait()
                _mkidx(g + 1, npar)
                _fire_pool(g + 1, npar)

                @pl.when(g + 2 < NG)
                def _():
                    _fire_ids(g + 2, par)

            mg = len_v[pl.ds(g * CQ, CQ)][CQ - 1]
            for l in range(L):
                @pl.when(l < mg)
                def _():
                    _pool_copy(par, l).wait()

            @pl.when(g >= 2)
            def _():
                pltpu.make_async_copy(
                    outc_v.at[par],
                    out_hbm.at[perm_v.at[pl.ds((g - 2) * CQ, CQ)]],
                    osems[par]).wait()

            len16c = len_v[pl.ds(g * CQ, CQ)]
            for p in range(CQ):
                lb = _take(len16c, jnp.full((LANES,), p, jnp.int32))
                sc = 1.0 / jnp.maximum(lb, 1).astype(jnp.float32)
                for cc in range(D // LANES):
                    v = acc_v[par, p, pl.ds(cc * LANES, LANES)]
                    outc_v[par, p, pl.ds(cc * LANES, LANES)] = v * sc
                    acc_v[par, p, pl.ds(cc * LANES, LANES)] = zero16

            pltpu.async_copy(outc_v.at[par],
                             out_hbm.at[perm_v.at[pl.ds(g * CQ, CQ)]],
                             osems[par])

    for b in range(2):
        g = NG - 2 + b
        pltpu.make_async_copy(outc_v.at[g % 2],
                              out_hbm.at[perm_v.at[pl.ds(g * CQ, CQ)]],
                              osems[g % 2]).wait()


@jax.jit
def _pooled(ids_t, lens_flat, perm_flat, table_blk):
    mesh = plsc.VectorSubcoreMesh(core_axis_name="c", subcore_axis_name="s")
    return pl.kernel(
        _body,
        out_type=jax.ShapeDtypeStruct((N, D), jnp.float32),
        mesh=mesh,
        scratch_types=[
            pltpu.VMEM((PW,), jnp.int32),
            pltpu.VMEM((PW,), jnp.int32),
            pltpu.VMEM((2, L, CQ), jnp.int32),
            pltpu.VMEM((2, L, CQ), jnp.int32),
            pltpu.VMEM((2, CQ, D), jnp.float32),
            pltpu.VMEM((2, CQ, D), jnp.float32),
            pltpu.VMEM_SHARED((SROWS, D), jnp.float32),
            [pltpu.SemaphoreType.DMA] * 2,
            [pltpu.SemaphoreType.DMA] * 2,
            [pltpu.SemaphoreType.DMA] * 2,
        ],
    )(ids_t, lens_flat, perm_flat, table_blk)


def kernel(dynamic_ids, dynamic_lengths, tables):
    M = N // 2
    lens2 = dynamic_lengths.astype(jnp.int32).T.reshape(2, M)
    perm2 = jnp.argsort(lens2, axis=1)
    lens_s = jnp.take_along_axis(lens2, perm2, axis=1)
    idsf = dynamic_ids.astype(jnp.int32).transpose(1, 0, 2).reshape(2, M, L)
    ids_s = jnp.take_along_axis(idsf, perm2[:, :, None], axis=1)
    ids_t = ids_s.reshape(N // CQ, CQ, L).transpose(0, 2, 1)
    lens_flat = lens_s.reshape(N)
    perm_flat = (perm2 + jnp.arange(2, dtype=jnp.int32)[:, None] * M
                 ).reshape(N).astype(jnp.int32)
    blk = tables.reshape(2, FH * (V + 1), D)
    blk = jnp.pad(blk, ((0, 0), (0, SROWS - FH * (V + 1)), (0, 0)))
    out = _pooled(ids_t, lens_flat, perm_flat, blk)
    return out.reshape(F, B, D).transpose(1, 0, 2)


# speedup vs baseline: 1.1922x; 1.1922x over previous
import jax
import jax.numpy as jnp
from jax import lax
from jax.experimental import pallas as pl
from jax.experimental.pallas import tpu as pltpu
from jax.experimental.pallas import tpu_sc as plsc

B, F, L, D, V = 1024, 26, 20, 128, 1000
N = B * F
NW = 32
PW = N // NW
CQ = 16
IDS = CQ * L
NG = PW // CQ
LANES = 16
FH = F // 2
ZROW = FH * (V + 1)
STG = 816
SROWS = 16 * STG


def _take(vec, idx):
    dnums = lax.GatherDimensionNumbers(
        offset_dims=(), collapsed_slice_dims=(0,), start_index_map=(0,))
    return lax.gather(vec, idx[:, None], dnums, (1,),
                      mode=lax.GatherScatterMode.PROMISE_IN_BOUNDS)


def _body(ids_hbm, len_hbm, table_hbm, out_hbm,
          len_v, idsb_v, idx_v, acc_v, outc_v, shared_v,
          isems, gsems, osems):
    c = lax.axis_index("c")
    s = lax.axis_index("s")
    wid = c * 16 + s
    pstart = wid * PW

    pltpu.sync_copy(table_hbm.at[c, pl.ds(s * STG, STG)],
                    shared_v.at[pl.ds(s * STG, STG)])
    plsc.subcore_barrier()

    pltpu.sync_copy(len_hbm.at[pl.ds(pstart, PW)], len_v.at[pl.ds(0, PW)])

    lane = lax.iota(jnp.int32, LANES)
    zero16 = jnp.zeros((LANES,), jnp.float32)

    for par in range(2):
        for p in range(CQ):
            for cc in range(D // LANES):
                acc_v[par, p, pl.ds(cc * LANES, LANES)] = zero16

    kbase = wid * NG

    def _fire_ids(g, b):
        pltpu.async_copy(ids_hbm.at[kbase + g], idsb_v.at[b], isems[b])

    def _mkidx(g, b):
        len16 = len_v[pl.ds(g * CQ, CQ)]
        n16 = (pstart + g * CQ) + lane
        base16 = (lax.shift_right_logical(n16, 10) - c * FH) * (V + 1)
        for l in range(L):
            id16 = idsb_v[b, l]
            idx = jnp.where(len16 > l, base16 + id16, ZROW)
            idx_v[b, l] = idx

    def _pool_copy(par, l):
        return pltpu.make_async_copy(
            shared_v.at[idx_v.at[par, l]], acc_v.at[par], gsems[par])

    def _fire_pool(par):
        for l in range(L):
            pltpu.async_copy(
                shared_v.at[idx_v.at[par, l]],
                acc_v.at[par], gsems[par], add=True)

    _fire_ids(0, 0)
    pltpu.make_async_copy(ids_hbm.at[kbase], idsb_v.at[0], isems[0]).wait()
    _mkidx(0, 0)
    _fire_ids(1, 1)
    _fire_pool(0)

    @pl.loop(0, NG, step=2)
    def _pool(g0):
        for par in range(2):
            g = g0 + par
            npar = (par + 1) % 2

            @pl.when(g + 1 < NG)
            def _():
                pltpu.make_async_copy(
                    ids_hbm.at[kbase + g + 1],
                    idsb_v.at[npar], isems[npar]).wait()
                _mkidx(g + 1, npar)
                _fire_pool(npar)

                @pl.when(g + 2 < NG)
                def _():
                    _fire_ids(g + 2, par)

            for l in range(L):
                _pool_copy(par, l).wait()

            @pl.when(g >= 2)
            def _():
                pltpu.make_async_copy(
                    outc_v.at[par],
                    out_hbm.at[pl.ds(pstart + (g - 2) * CQ, CQ)],
                    osems[par]).wait()

            len16c = len_v[pl.ds(g * CQ, CQ)]
            for p in range(CQ):
                lb = _take(len16c, jnp.full((LANES,), p, jnp.int32))
                sc = 1.0 / jnp.maximum(lb, 1).astype(jnp.float32)
                for cc in range(D // LANES):
                    v = acc_v[par, p, pl.ds(cc * LANES, LANES)]
                    outc_v[par, p, pl.ds(cc * LANES, LANES)] = v * sc
                    acc_v[par, p, pl.ds(cc * LANES, LANES)] = zero16

            pltpu.async_copy(outc_v.at[par],
                             out_hbm.at[pl.ds(pstart + g * CQ, CQ)],
                             osems[par])

    for b in range(2):
        g = NG - 2 + b
        pltpu.make_async_copy(outc_v.at[g % 2],
                              out_hbm.at[pl.ds(pstart + g * CQ, CQ)],
                              osems[g % 2]).wait()


@jax.jit
def _pooled(ids_t, lens_flat, table_blk):
    mesh = plsc.VectorSubcoreMesh(core_axis_name="c", subcore_axis_name="s")
    return pl.kernel(
        _body,
        out_type=jax.ShapeDtypeStruct((N, D), jnp.float32),
        mesh=mesh,
        scratch_types=[
            pltpu.VMEM((PW,), jnp.int32),
            pltpu.VMEM((2, L, CQ), jnp.int32),
            pltpu.VMEM((2, L, CQ), jnp.int32),
            pltpu.VMEM((2, CQ, D), jnp.float32),
            pltpu.VMEM((2, CQ, D), jnp.float32),
            pltpu.VMEM_SHARED((SROWS, D), jnp.float32),
            [pltpu.SemaphoreType.DMA] * 2,
            [pltpu.SemaphoreType.DMA] * 2,
            [pltpu.SemaphoreType.DMA] * 2,
        ],
    )(ids_t, lens_flat, table_blk)


def kernel(dynamic_ids, dynamic_lengths, tables):
    idsf = dynamic_ids.astype(jnp.int32).transpose(1, 0, 2).reshape(N, L)
    ids_t = idsf.reshape(N // CQ, CQ, L).transpose(0, 2, 1)
    lens_flat = dynamic_lengths.astype(jnp.int32).T.reshape(N)
    blk = tables.reshape(2, FH * (V + 1), D)
    blk = jnp.pad(blk, ((0, 0), (0, SROWS - FH * (V + 1)), (0, 0)))
    out = _pooled(ids_t, lens_flat, blk)
    return out.reshape(F, B, D).transpose(1, 0, 2)
